# baseline (speedup 1.0000x reference)
import jax
import jax.numpy as jnp
from jax import lax
from jax.experimental import pallas as pl
from jax.experimental.pallas import tpu as pltpu
from jax.experimental.pallas import tpu_sc as plsc

B = 4
H = 224
W = 224
HW = H * W
C = 96
NSEG = 100
NPIX = B * HW
NC = 2
NS = 16
NW = NC * NS
BPC = 2
HPW = H * BPC // NW
SCH = 112
SPERW = W // SCH
NIDXR = NPIX // SCH
IDXC = NIDXR // B * BPC
IPW = IDXC // NW
PARTS_PER_B = NW // BPC
ACC_ROWS = 512
HB = 32


def _prep_body(slic_ref, idx_ref):
    b = pl.program_id(0)
    x = slic_ref[0]
    v = jnp.where(x == 0, 400 + b, x * 4 + (b - 4))
    rows = []
    for r in range(HB * W // SCH):
        h0, o0 = divmod(r * SCH, W)
        rows.append(v[h0, o0:o0 + SCH])
    idx_ref[...] = jnp.stack(rows, axis=0)


@jax.jit
def _prep_call(slic3d):
    return pl.pallas_call(
        _prep_body,
        grid=(B, H // HB),
        in_specs=[pl.BlockSpec((1, HB, W), lambda b, h: (b, h, 0))],
        out_specs=pl.BlockSpec((HB * W // SCH, SCH),
                               lambda b, h: (b * (H // HB) + h, 0)),
        out_shape=jax.ShapeDtypeStruct((NIDXR, SCH), jnp.int32),
    )(slic3d)


def _sc_body(img_hbm, idx_hbm, psum_hbm, pcnt_hbm,
             idx_v, img_v, ind_v, ones_v, zacc_v, acc_s, acc_c,
             sem_in, sem_sum, sem_cnt):
    c = lax.axis_index("c")
    s = lax.axis_index("s")
    wid = s * NC + c
    b = wid // PARTS_PER_B
    h_base = (wid % PARTS_PER_B) * HPW

    row_start = pl.multiple_of(lax.bitwise_and(wid * IPW, ~7), 8)
    roff = wid * IPW - row_start
    pltpu.sync_copy(idx_hbm.at[pl.ds(row_start, IPW + 4)], idx_v)

    one = jnp.full((16,), 1.0, jnp.float32)
    zero = jnp.zeros((16,), jnp.float32)

    def fill_body(r, _):
        for k in range(C // 16):
            img_v[0, r, pl.ds(k * 16, 16)] = zero
        return 0
    lax.fori_loop(0, 128, fill_body, 0)

    def ones_body(r, _):
        for k in range(C // 16):
            ones_v[r, pl.ds(k * 16, 16)] = one
        return 0
    lax.fori_loop(0, SCH, ones_body, 0)

    @pl.when(s < 8)
    def _():
        row0 = pl.multiple_of((s % 4) * 128, 128)

        @pl.when(s < 4)
        def _():
            pltpu.sync_copy(img_v.at[0, pl.ds(0, 128)],
                            acc_s.at[pl.ds(row0, 128)])

        @pl.when(s >= 4)
        def _():
            pltpu.sync_copy(img_v.at[0, pl.ds(0, 128)],
                            acc_c.at[pl.ds(row0, 128)])

    plsc.subcore_barrier()

    pltpu.async_copy(img_hbm.at[b, h_base], img_v.at[0], sem_in.at[0])

    def slab_body(j, npend):
        buf = j % 2
        nbuf = (j + 1) % 2

        pltpu.make_async_copy(img_hbm.at[b, h_base], img_v.at[buf],
                              sem_in.at[buf]).wait()

        for k in range(SPERW):
            pltpu.async_copy(img_v.at[buf, pl.ds(k * SCH, SCH)],
                            acc_s.at[idx_v.at[roff + j * SPERW + k]],
                            sem_sum.at[buf, k], add=True)

        @pl.when(j >= 1)
        def _():
            for k in range(SPERW):
                pltpu.make_async_copy(
                    img_v.at[nbuf, pl.ds(k * SCH, SCH)],
                    acc_s.at[idx_v.at[roff + (j - 1) * SPERW + k]],
                    sem_sum.at[nbuf, k]).wait()

        @pl.when(j < HPW - 1)
        def _():
            pltpu.async_copy(img_hbm.at[b, h_base + j + 1],
                             img_v.at[nbuf], sem_in.at[nbuf])

        zacc_v[pl.ds(0, 16)] = jnp.zeros((16,), jnp.int32)

        def z_body(i, _):
            r0 = i * 8
            zacc = zacc_v[pl.ds(0, 16)]
            for dr in range(8):
                for k in range(C // 16):
                    x = img_v[buf, r0 + dr, pl.ds(k * 16, 16)]
                    zacc = zacc | jnp.where(x == 0.0, 1, 0)
            zacc_v[pl.ds(0, 16)] = zacc
            return 0
        lax.fori_loop(0, W // 8, z_body, 0)
        nz = zacc_v[pl.ds(0, 16)]
        any_zero = nz[0]
        for l in range(1, 16):
            any_zero = any_zero | nz[l]

        @pl.when(any_zero == 0)
        def _():
            for k in range(SPERW):
                pltpu.async_copy(ones_v,
                                 acc_c.at[idx_v.at[roff + j * SPERW + k]],
                                 sem_cnt, add=True)

        @pl.when(any_zero > 0)
        def _():
            def ind_body(r, _):
                for k in range(C // 16):
                    x = img_v[buf, r, pl.ds(k * 16, 16)]
                    ind_v[r, pl.ds(k * 16, 16)] = jnp.where(
                        x != 0.0, 1.0, 0.0).astype(jnp.float32)
                return 0
            lax.fori_loop(0, W, ind_body, 0)
            for k in range(SPERW):
                pltpu.sync_copy(ind_v.at[pl.ds(k * SCH, SCH)],
                                acc_c.at[idx_v.at[roff + j * SPERW + k]], add=True)

        return npend + jnp.where(any_zero == 0, 2, 0)

    npend = lax.fori_loop(0, HPW, slab_body, 0)

    for k in range(SPERW):
        pltpu.make_async_copy(
            img_v.at[(HPW - 1) % 2, pl.ds(k * SCH, SCH)],
            acc_s.at[idx_v.at[roff + (HPW - 1) * SPERW + k]],
            sem_sum.at[(HPW - 1) % 2, k]).wait()

    def drain_body(i, _):
        pltpu.make_async_copy(ones_v, acc_c.at[idx_v.at[roff]], sem_cnt).wait()
        return 0
    lax.fori_loop(0, npend, drain_body, 0)

    plsc.subcore_barrier()

    @pl.when(s == 0)
    def _():
        pltpu.sync_copy(acc_s, psum_hbm.at[c])
        pltpu.sync_copy(acc_c, pcnt_hbm.at[c])


@jax.jit
def _sc_call(img4d, idx2d):
    mesh = plsc.VectorSubcoreMesh(core_axis_name="c", subcore_axis_name="s")
    f = pl.kernel(
        _sc_body,
        out_type=(
            jax.ShapeDtypeStruct((NC, ACC_ROWS, C), jnp.float32),
            jax.ShapeDtypeStruct((NC, ACC_ROWS, C), jnp.float32),
        ),
        mesh=mesh,
        compiler_params=pltpu.CompilerParams(use_tc_tiling_on_sc=True),
        scratch_types=[
            pltpu.VMEM((IPW + 4, SCH), jnp.int32),
            pltpu.VMEM((2, W, C), jnp.float32),
            pltpu.VMEM((W, C), jnp.float32),
            pltpu.VMEM((SCH, C), jnp.float32),
            pltpu.VMEM((16,), jnp.int32),
            pltpu.VMEM_SHARED((ACC_ROWS, C), jnp.float32),
            pltpu.VMEM_SHARED((ACC_ROWS, C), jnp.float32),
            pltpu.SemaphoreType.DMA((2,)),
            pltpu.SemaphoreType.DMA((2, SPERW)),
            pltpu.SemaphoreType.DMA,
        ],
    )
    return f(img4d, idx2d)


def _combine_body(psa_ref, pca_ref, psb_ref, pcb_ref, o_ref):
    ssum = (psa_ref[0, 0:NSEG * B, :] + psa_ref[1, 0:NSEG * B, :]
            + psb_ref[0, 0:NSEG * B, :] + psb_ref[1, 0:NSEG * B, :])
    scnt = (pca_ref[0, 0:NSEG * B, :] + pca_ref[1, 0:NSEG * B, :]
            + pcb_ref[0, 0:NSEG * B, :] + pcb_ref[1, 0:NSEG * B, :])
    o_ref[...] = ssum / scnt


@jax.jit
def _combine_call(psa, pca, psb, pcb):
    return pl.pallas_call(
        _combine_body,
        out_shape=jax.ShapeDtypeStruct((NSEG * B, C), jnp.float32),
    )(psa, pca, psb, pcb)


def kernel(image_output, slic_output):
    idx2d = _prep_call(jnp.squeeze(slic_output, -1))
    psa, pca = _sc_call(image_output[0:BPC], idx2d[0:IDXC])
    psb, pcb = _sc_call(image_output[BPC:B], idx2d[IDXC:NIDXR])
    out2d = _combine_call(psa, pca, psb, pcb)
    return out2d.reshape(NSEG, B, C)


# speedup vs baseline: 1.3034x; 1.3034x over previous
import jax
import jax.numpy as jnp
from jax import lax
from jax.experimental import pallas as pl
from jax.experimental.pallas import tpu as pltpu
from jax.experimental.pallas import tpu_sc as plsc

B = 4
H = 224
W = 224
HW = H * W
C = 96
NSEG = 100
NPIX = B * HW
NC = 2
NS = 16
NW = NC * NS
HPW = H * B // NW
SCH = 112
SPERW = W // SCH
NIDXR = NPIX // SCH
IPW = NIDXR // NW
PARTS_PER_B = NW // B
ACC_ROWS = 512
HB = 32


def _sc_body(img_hbm, slic_hbm, psum_hbm, pcnt_hbm,
             slic_v, idx_v, img_v, ind_v, ones_v, zacc_v, acc_s, acc_c,
             sem_in, sem_sum, sem_cnt):
    c = lax.axis_index("c")
    s = lax.axis_index("s")
    wid = s * NC + c
    b = wid // PARTS_PER_B
    h_base = (wid % PARTS_PER_B) * HPW

    h0 = pl.multiple_of(lax.bitwise_and(h_base, ~7), 8)
    hoff = h_base - h0
    pltpu.sync_copy(slic_hbm.at[b, pl.ds(h0, HPW + 4)], slic_v)

    def idx_body(j, _):
        for k in range(SPERW):
            for t in range(SCH // 16):
                v = slic_v[hoff + j, pl.ds(k * SCH + t * 16, 16)]
                ix = jnp.where(v == 0, 400 + b, v * 4 + (b - 4))
                idx_v[j * SPERW + k, pl.ds(t * 16, 16)] = ix
        return 0
    lax.fori_loop(0, HPW, idx_body, 0)

    one = jnp.full((16,), 1.0, jnp.float32)
    zero = jnp.zeros((16,), jnp.float32)

    def fill_body(r, _):
        for k in range(C // 16):
            img_v[0, r, pl.ds(k * 16, 16)] = zero
        return 0
    lax.fori_loop(0, 128, fill_body, 0)

    def ones_body(r, _):
        for k in range(C // 16):
            ones_v[r, pl.ds(k * 16, 16)] = one
        return 0
    lax.fori_loop(0, SCH, ones_body, 0)

    @pl.when(s < 8)
    def _():
        row0 = pl.multiple_of((s % 4) * 128, 128)

        @pl.when(s < 4)
        def _():
            pltpu.sync_copy(img_v.at[0, pl.ds(0, 128)],
                            acc_s.at[pl.ds(row0, 128)])

        @pl.when(s >= 4)
        def _():
            pltpu.sync_copy(img_v.at[0, pl.ds(0, 128)],
                            acc_c.at[pl.ds(row0, 128)])

    plsc.subcore_barrier()

    pltpu.async_copy(img_hbm.at[b, h_base], img_v.at[0], sem_in.at[0])

    def slab_body(j, npend):
        buf = j % 2
        nbuf = (j + 1) % 2

        pltpu.make_async_copy(img_hbm.at[b, h_base], img_v.at[buf],
                              sem_in.at[buf]).wait()

        for k in range(SPERW):
            pltpu.async_copy(img_v.at[buf, pl.ds(k * SCH, SCH)],
                            acc_s.at[idx_v.at[j * SPERW + k]],
                            sem_sum.at[buf, k], add=True)

        @pl.when(j >= 1)
        def _():
            for k in range(SPERW):
                pltpu.make_async_copy(
                    img_v.at[nbuf, pl.ds(k * SCH, SCH)],
                    acc_s.at[idx_v.at[(j - 1) * SPERW + k]],
                    sem_sum.at[nbuf, k]).wait()

        @pl.when(j < HPW - 1)
        def _():
            pltpu.async_copy(img_hbm.at[b, h_base + j + 1],
                             img_v.at[nbuf], sem_in.at[nbuf])

        zacc_v[pl.ds(0, 16)] = jnp.zeros((16,), jnp.int32)

        def z_body(i, _):
            r0 = i * 8
            zacc = zacc_v[pl.ds(0, 16)]
            for dr in range(8):
                for k in range(C // 16):
                    x = img_v[buf, r0 + dr, pl.ds(k * 16, 16)]
                    zacc = zacc | jnp.where(x == 0.0, 1, 0)
            zacc_v[pl.ds(0, 16)] = zacc
            return 0
        lax.fori_loop(0, W // 8, z_body, 0)
        nz = zacc_v[pl.ds(0, 16)]
        any_zero = nz[0]
        for l in range(1, 16):
            any_zero = any_zero | nz[l]

        @pl.when(any_zero == 0)
        def _():
            for k in range(SPERW):
                pltpu.async_copy(ones_v,
                                 acc_c.at[idx_v.at[j * SPERW + k]],
                                 sem_cnt, add=True)

        @pl.when(any_zero > 0)
        def _():
            def ind_body(r, _):
                for k in range(C // 16):
                    x = img_v[buf, r, pl.ds(k * 16, 16)]
                    ind_v[r, pl.ds(k * 16, 16)] = jnp.where(
                        x != 0.0, 1.0, 0.0).astype(jnp.float32)
                return 0
            lax.fori_loop(0, W, ind_body, 0)
            for k in range(SPERW):
                pltpu.sync_copy(ind_v.at[pl.ds(k * SCH, SCH)],
                                acc_c.at[idx_v.at[j * SPERW + k]], add=True)

        return npend + jnp.where(any_zero == 0, 2, 0)

    npend = lax.fori_loop(0, HPW, slab_body, 0)

    for k in range(SPERW):
        pltpu.make_async_copy(
            img_v.at[(HPW - 1) % 2, pl.ds(k * SCH, SCH)],
            acc_s.at[idx_v.at[(HPW - 1) * SPERW + k]],
            sem_sum.at[(HPW - 1) % 2, k]).wait()

    def drain_body(i, _):
        pltpu.make_async_copy(ones_v, acc_c.at[idx_v.at[0]], sem_cnt).wait()
        return 0
    lax.fori_loop(0, npend, drain_body, 0)

    plsc.subcore_barrier()

    @pl.when(s == 0)
    def _():
        pltpu.sync_copy(acc_s, psum_hbm.at[c])
        pltpu.sync_copy(acc_c, pcnt_hbm.at[c])


@jax.jit
def _sc_call(img4d, slic3d):
    mesh = plsc.VectorSubcoreMesh(core_axis_name="c", subcore_axis_name="s")
    f = pl.kernel(
        _sc_body,
        out_type=(
            jax.ShapeDtypeStruct((NC, ACC_ROWS, C), jnp.float32),
            jax.ShapeDtypeStruct((NC, ACC_ROWS, C), jnp.float32),
        ),
        mesh=mesh,
        compiler_params=pltpu.CompilerParams(use_tc_tiling_on_sc=True),
        scratch_types=[
            pltpu.VMEM((HPW + 4, W), jnp.int32),
            pltpu.VMEM((IPW, SCH), jnp.int32),
            pltpu.VMEM((2, W, C), jnp.float32),
            pltpu.VMEM((W, C), jnp.float32),
            pltpu.VMEM((SCH, C), jnp.float32),
            pltpu.VMEM((16,), jnp.int32),
            pltpu.VMEM_SHARED((ACC_ROWS, C), jnp.float32),
            pltpu.VMEM_SHARED((ACC_ROWS, C), jnp.float32),
            pltpu.SemaphoreType.DMA((2,)),
            pltpu.SemaphoreType.DMA((2, SPERW)),
            pltpu.SemaphoreType.DMA,
        ],
    )
    return f(img4d, slic3d)


def _combine_body(ps_ref, pc_ref, o_ref):
    ssum = ps_ref[0, 0:NSEG * B, :] + ps_ref[1, 0:NSEG * B, :]
    scnt = pc_ref[0, 0:NSEG * B, :] + pc_ref[1, 0:NSEG * B, :]
    o_ref[...] = ssum / scnt


@jax.jit
def _combine_call(psum, pcnt):
    return pl.pallas_call(
        _combine_body,
        out_shape=jax.ShapeDtypeStruct((NSEG * B, C), jnp.float32),
    )(psum, pcnt)


def kernel(image_output, slic_output):
    psum, pcnt = _sc_call(image_output, jnp.squeeze(slic_output, -1))
    out2d = _combine_call(psum, pcnt)
    return out2d.reshape(NSEG, B, C)
